# speedup vs baseline: 4.5488x; 4.5488x over previous
import jax
import jax.numpy as jnp
from jax.experimental import pallas as pl
from jax.experimental.pallas import tpu as pltpu

_BATCH = 16384
_FDIM = 2048
_BR = 512


def _body(idx_ref, x_ref, o_ref, mask_ref):
    @pl.when(pl.program_id(0) == 0)
    def _():
        ones = jnp.ones((1, _FDIM), jnp.float32)
        iota = jax.lax.broadcasted_iota(jnp.int32, (1, _FDIM), 1)

        def upd(i, m):
            return jnp.where(iota == idx_ref[i], 0.0, m)

        mask_ref[...] = jax.lax.fori_loop(0, idx_ref.shape[0], upd, ones)

    o_ref[...] = x_ref[...] * mask_ref[...]


def kernel(x, mask_indices):
    grid = (_BATCH // _BR,)
    return pl.pallas_call(
        _body,
        grid_spec=pltpu.PrefetchScalarGridSpec(
            num_scalar_prefetch=1,
            grid=grid,
            in_specs=[pl.BlockSpec((_BR, _FDIM), lambda i, *_: (i, 0))],
            out_specs=pl.BlockSpec((_BR, _FDIM), lambda i, *_: (i, 0)),
            scratch_shapes=[pltpu.VMEM((1, _FDIM), jnp.float32)],
        ),
        out_shape=jax.ShapeDtypeStruct((_BATCH, _FDIM), jnp.float32),
        compiler_params=pltpu.CompilerParams(
            dimension_semantics=("arbitrary",),
        ),
    )(mask_indices, x)


# speedup vs baseline: 4.6550x; 1.0233x over previous
import jax
import jax.numpy as jnp
from jax.experimental import pallas as pl
from jax.experimental.pallas import tpu as pltpu

_BATCH = 16384
_FDIM = 2048
_BR = 1024


def _body(idx_ref, x_ref, o_ref, mask_ref):
    @pl.when(pl.program_id(0) == 0)
    def _():
        ones = jnp.ones((1, _FDIM), jnp.float32)
        iota = jax.lax.broadcasted_iota(jnp.int32, (1, _FDIM), 1)

        def upd(i, m):
            return jnp.where(iota == idx_ref[i], 0.0, m)

        mask_ref[...] = jax.lax.fori_loop(0, idx_ref.shape[0], upd, ones)

    o_ref[...] = x_ref[...] * mask_ref[...]


def kernel(x, mask_indices):
    grid = (_BATCH // _BR,)
    return pl.pallas_call(
        _body,
        grid_spec=pltpu.PrefetchScalarGridSpec(
            num_scalar_prefetch=1,
            grid=grid,
            in_specs=[pl.BlockSpec((_BR, _FDIM), lambda i, *_: (i, 0))],
            out_specs=pl.BlockSpec((_BR, _FDIM), lambda i, *_: (i, 0)),
            scratch_shapes=[pltpu.VMEM((1, _FDIM), jnp.float32)],
        ),
        out_shape=jax.ShapeDtypeStruct((_BATCH, _FDIM), jnp.float32),
        compiler_params=pltpu.CompilerParams(
            dimension_semantics=("arbitrary",),
        ),
    )(mask_indices, x)
